# Initial kernel scaffold; baseline (speedup 1.0000x reference)
import jax
import jax.numpy as jnp
from jax.experimental import pallas as pl


def kernel(x, edge_index, norm, graph_ids, W, b, gamma, beta):
    raise NotImplementedError("write your pallas kernel here")


# speedup vs baseline: 3.2909x; 3.2909x over previous
import functools

import jax
import jax.numpy as jnp
from jax import lax
from jax.experimental import pallas as pl
from jax.experimental.pallas import tpu as pltpu
from jax.experimental.pallas import tpu_sc as plsc

N_NODES = 10000
N_EDGES = 320000
D = 128
NUM_GRAPHS = 64

NC, NS, L = 2, 16, 16
NW = NC * NS
CHUNK = 128
CHUNKS_PER_W = 80
EDGES_PER_W = CHUNK * CHUNKS_PER_W
E_PAD = NW * EDGES_PER_W
ROWS_PER_TILE = 640
ACC_ROWS = ROWS_PER_TILE * NS
DUMP_ROW = N_NODES

BLK = 1000
NBLK = N_NODES // BLK


def _scale_body(x_ref, norm_ref, o_ref):
    o_ref[...] = x_ref[...] * norm_ref[...]


def _edge_agg_body(h1_hbm, src_hbm, dst_hbm, out_hbm,
                   src_v, dst_v, rows_v, zero_v, acc_sh):
    c = lax.axis_index("c")
    s = lax.axis_index("s")
    wid = c * NS + s

    zvec = jnp.zeros((L,), jnp.float32)
    for r in range(16):
        for q in range(D // L):
            zero_v[r, pl.ds(q * L, L)] = zvec

    def zbody(i, carry):
        pltpu.sync_copy(zero_v,
                        acc_sh.at[pl.ds(s * ROWS_PER_TILE + i * 16, 16)])
        return carry
    lax.fori_loop(0, ROWS_PER_TILE // 16, zbody, 0)

    pltpu.sync_copy(src_hbm.at[pl.ds(wid * CHUNKS_PER_W, CHUNKS_PER_W)], src_v)
    pltpu.sync_copy(dst_hbm.at[pl.ds(wid * CHUNKS_PER_W, CHUNKS_PER_W)], dst_v)

    plsc.subcore_barrier()

    def body(j, carry):
        pltpu.sync_copy(h1_hbm.at[src_v.at[j]], rows_v)
        pltpu.sync_copy(rows_v, acc_sh.at[dst_v.at[j]], add=True)
        return carry
    lax.fori_loop(0, CHUNKS_PER_W, body, 0)

    plsc.subcore_barrier()

    pltpu.sync_copy(acc_sh.at[pl.ds(s * ROWS_PER_TILE, ROWS_PER_TILE)],
                    out_hbm.at[c, pl.ds(s * ROWS_PER_TILE, ROWS_PER_TILE)])


def _fc_body(p_ref, norm_ref, w_ref, b_ref, h3_ref, sum_ref, sumsq_ref):
    i = pl.program_id(0)
    agg = p_ref[0] + p_ref[1]
    h = agg * norm_ref[...]
    h = lax.dot_general(h, w_ref[...], (((1,), (1,)), ((), ())),
                        precision=lax.Precision.HIGHEST,
                        preferred_element_type=jnp.float32)
    h = jnp.maximum(h + b_ref[...], 0.0)
    h3_ref[...] = h

    @pl.when(i == 0)
    def _():
        sum_ref[...] = jnp.zeros_like(sum_ref)
        sumsq_ref[...] = jnp.zeros_like(sumsq_ref)

    sum_ref[...] += jnp.sum(h, axis=0, keepdims=True)
    sumsq_ref[...] += jnp.sum(h * h, axis=0, keepdims=True)


def _bn_body(h3_ref, sum_ref, sumsq_ref, gamma_ref, beta_ref, gid_ref,
             hbn_ref, phis_ref):
    i = pl.program_id(0)
    inv_n = 1.0 / N_NODES
    mean = sum_ref[...] * inv_n
    var = sumsq_ref[...] * inv_n - mean * mean
    scale = gamma_ref[...] / jnp.sqrt(var + 1e-5)
    hbn = (h3_ref[...] - mean) * scale + beta_ref[...]
    hbn_ref[...] = hbn

    gid = gid_ref[0]
    gids = lax.broadcasted_iota(jnp.int32, (NUM_GRAPHS, BLK), 0)
    onehot = (gids == gid).astype(jnp.float32)
    contrib = lax.dot_general(onehot, hbn, (((1,), (0,)), ((), ())),
                              precision=lax.Precision.HIGHEST,
                              preferred_element_type=jnp.float32)

    @pl.when(i == 0)
    def _():
        phis_ref[...] = jnp.zeros_like(phis_ref)

    phis_ref[...] += contrib


def kernel(x, edge_index, norm, graph_ids, W, b, gamma, beta):
    h1 = pl.pallas_call(
        _scale_body,
        grid=(NBLK,),
        in_specs=[pl.BlockSpec((BLK, D), lambda i: (i, 0)),
                  pl.BlockSpec((BLK, 1), lambda i: (i, 0))],
        out_specs=pl.BlockSpec((BLK, D), lambda i: (i, 0)),
        out_shape=jax.ShapeDtypeStruct((N_NODES, D), jnp.float32),
    )(x, norm)

    pad = E_PAD - N_EDGES
    src2d = jnp.concatenate(
        [edge_index[0], jnp.zeros((pad,), jnp.int32)]).reshape(-1, CHUNK)
    dst2d = jnp.concatenate(
        [edge_index[1], jnp.full((pad,), DUMP_ROW, jnp.int32)]).reshape(-1, CHUNK)

    mesh = plsc.VectorSubcoreMesh(core_axis_name="c", subcore_axis_name="s",
                                  num_cores=NC, num_subcores=NS)
    partials = pl.kernel(
        _edge_agg_body,
        out_type=jax.ShapeDtypeStruct((NC, ACC_ROWS, D), jnp.float32),
        mesh=mesh,
        scratch_types=[
            pltpu.VMEM((CHUNKS_PER_W, CHUNK), jnp.int32),
            pltpu.VMEM((CHUNKS_PER_W, CHUNK), jnp.int32),
            pltpu.VMEM((CHUNK, D), jnp.float32),
            pltpu.VMEM((16, D), jnp.float32),
            pltpu.VMEM_SHARED((ACC_ROWS, D), jnp.float32),
        ],
    )(h1, src2d, dst2d)

    p = partials[:, :N_NODES, :]

    h3, colsum, colsumsq = pl.pallas_call(
        _fc_body,
        grid=(NBLK,),
        in_specs=[pl.BlockSpec((NC, BLK, D), lambda i: (0, i, 0)),
                  pl.BlockSpec((BLK, 1), lambda i: (i, 0)),
                  pl.BlockSpec((D, D), lambda i: (0, 0)),
                  pl.BlockSpec((1, D), lambda i: (0, 0))],
        out_specs=[pl.BlockSpec((BLK, D), lambda i: (i, 0)),
                   pl.BlockSpec((1, D), lambda i: (0, 0)),
                   pl.BlockSpec((1, D), lambda i: (0, 0))],
        out_shape=[jax.ShapeDtypeStruct((N_NODES, D), jnp.float32),
                   jax.ShapeDtypeStruct((1, D), jnp.float32),
                   jax.ShapeDtypeStruct((1, D), jnp.float32)],
    )(p, norm, W, b.reshape(1, D))

    gid3 = graph_ids.reshape(NBLK, 1, BLK)
    hbn, phis = pl.pallas_call(
        _bn_body,
        grid=(NBLK,),
        in_specs=[pl.BlockSpec((BLK, D), lambda i: (i, 0)),
                  pl.BlockSpec((1, D), lambda i: (0, 0)),
                  pl.BlockSpec((1, D), lambda i: (0, 0)),
                  pl.BlockSpec((1, D), lambda i: (0, 0)),
                  pl.BlockSpec((1, D), lambda i: (0, 0)),
                  pl.BlockSpec((1, 1, BLK), lambda i: (i, 0, 0))],
        out_specs=[pl.BlockSpec((BLK, D), lambda i: (i, 0)),
                   pl.BlockSpec((NUM_GRAPHS, D), lambda i: (0, 0))],
        out_shape=[jax.ShapeDtypeStruct((N_NODES, D), jnp.float32),
                   jax.ShapeDtypeStruct((NUM_GRAPHS, D), jnp.float32)],
    )(h3, colsum, colsumsq, gamma.reshape(1, D), beta.reshape(1, D), gid3)

    return (hbn, phis)


# speedup vs baseline: 3.5467x; 1.0777x over previous
import functools

import jax
import jax.numpy as jnp
from jax import lax
from jax.experimental import pallas as pl
from jax.experimental.pallas import tpu as pltpu
from jax.experimental.pallas import tpu_sc as plsc

N_NODES = 10000
N_EDGES = 320000
D = 128
NUM_GRAPHS = 64

NC, NS, L = 2, 16, 16
NW = NC * NS
CHUNK = 128
CHUNKS_PER_W = 80
HALF = CHUNKS_PER_W // 2
EDGES_PER_W = CHUNK * CHUNKS_PER_W
E_PAD = NW * EDGES_PER_W
ROWS_PER_TILE = 632
ACC_ROWS = ROWS_PER_TILE * NS
DUMP_ROW = N_NODES

BLK = 1000
NBLK = N_NODES // BLK


def _scale_body(x_ref, norm_ref, o_ref):
    o_ref[...] = x_ref[...] * norm_ref[...]


def _edge_agg_body(h1_hbm, src_hbm, dst_hbm, out_hbm,
                   src_v, dst_v, rows_v, zero_v, acc_sh,
                   gsem0, gsem1, ssem0, ssem1):
    c = lax.axis_index("c")
    s = lax.axis_index("s")
    wid = c * NS + s

    zvec = jnp.zeros((L,), jnp.float32)
    for r in range(16):
        for q in range(D // L):
            zero_v[r, pl.ds(q * L, L)] = zvec

    def zbody(i, carry):
        pltpu.sync_copy(zero_v,
                        acc_sh.at[pl.ds(s * ROWS_PER_TILE + i * 16, 16)])
        return carry
    lax.fori_loop(0, ROWS_PER_TILE // 16, zbody, 0)
    pltpu.sync_copy(
        zero_v.at[pl.ds(0, ROWS_PER_TILE % 16)],
        acc_sh.at[pl.ds(s * ROWS_PER_TILE + 16 * (ROWS_PER_TILE // 16),
                        ROWS_PER_TILE % 16)])

    def fire_gather(g, p, sem):
        pltpu.async_copy(h1_hbm.at[src_v.at[g]], rows_v.at[p], sem)

    def drain_gather(p, sem):
        pltpu.make_async_copy(h1_hbm.at[pl.ds(0, CHUNK)],
                              rows_v.at[p], sem).wait()

    def fire_scatter(g, p, sem):
        pltpu.async_copy(rows_v.at[p], acc_sh.at[dst_v.at[g]], sem, add=True)

    def drain_scatter(p, sem):
        pltpu.make_async_copy(rows_v.at[p],
                              acc_sh.at[pl.ds(0, CHUNK)], sem).wait()

    plsc.subcore_barrier()

    for h in range(2):
        pltpu.sync_copy(
            src_hbm.at[pl.ds(wid * CHUNKS_PER_W + h * HALF, HALF)], src_v)
        pltpu.sync_copy(
            dst_hbm.at[pl.ds(wid * CHUNKS_PER_W + h * HALF, HALF)], dst_v)

        fire_gather(0, 0, gsem0)
        fire_gather(1, 1, gsem1)

        def body(i, carry):
            g0 = 2 * i
            drain_gather(0, gsem0)
            fire_scatter(g0, 0, ssem0)
            drain_gather(1, gsem1)
            drain_scatter(0, ssem0)

            @pl.when(i < HALF // 2 - 1)
            def _():
                fire_gather(g0 + 2, 0, gsem0)

            fire_scatter(g0 + 1, 1, ssem1)
            drain_scatter(1, ssem1)

            @pl.when(i < HALF // 2 - 1)
            def _():
                fire_gather(g0 + 3, 1, gsem1)

            return carry
        lax.fori_loop(0, HALF // 2, body, 0)

    plsc.subcore_barrier()

    pltpu.sync_copy(acc_sh.at[pl.ds(s * ROWS_PER_TILE, ROWS_PER_TILE)],
                    out_hbm.at[c, pl.ds(s * ROWS_PER_TILE, ROWS_PER_TILE)])


def _fc_body(p_ref, norm_ref, w_ref, b_ref, h3_ref, sum_ref, sumsq_ref):
    i = pl.program_id(0)
    agg = p_ref[0] + p_ref[1]
    h = agg * norm_ref[...]
    h = lax.dot_general(h, w_ref[...], (((1,), (1,)), ((), ())),
                        precision=lax.Precision.HIGHEST,
                        preferred_element_type=jnp.float32)
    h = jnp.maximum(h + b_ref[...], 0.0)
    h3_ref[...] = h

    @pl.when(i == 0)
    def _():
        sum_ref[...] = jnp.zeros_like(sum_ref)
        sumsq_ref[...] = jnp.zeros_like(sumsq_ref)

    sum_ref[...] += jnp.sum(h, axis=0, keepdims=True)
    sumsq_ref[...] += jnp.sum(h * h, axis=0, keepdims=True)


def _bn_body(h3_ref, sum_ref, sumsq_ref, gamma_ref, beta_ref, gid_ref,
             hbn_ref, phis_ref):
    i = pl.program_id(0)
    inv_n = 1.0 / N_NODES
    mean = sum_ref[...] * inv_n
    var = sumsq_ref[...] * inv_n - mean * mean
    scale = gamma_ref[...] / jnp.sqrt(var + 1e-5)
    hbn = (h3_ref[...] - mean) * scale + beta_ref[...]
    hbn_ref[...] = hbn

    gid = gid_ref[0]
    gids = lax.broadcasted_iota(jnp.int32, (NUM_GRAPHS, BLK), 0)
    onehot = (gids == gid).astype(jnp.float32)
    contrib = lax.dot_general(onehot, hbn, (((1,), (0,)), ((), ())),
                              precision=lax.Precision.HIGHEST,
                              preferred_element_type=jnp.float32)

    @pl.when(i == 0)
    def _():
        phis_ref[...] = jnp.zeros_like(phis_ref)

    phis_ref[...] += contrib


def kernel(x, edge_index, norm, graph_ids, W, b, gamma, beta):
    h1 = pl.pallas_call(
        _scale_body,
        grid=(NBLK,),
        in_specs=[pl.BlockSpec((BLK, D), lambda i: (i, 0)),
                  pl.BlockSpec((BLK, 1), lambda i: (i, 0))],
        out_specs=pl.BlockSpec((BLK, D), lambda i: (i, 0)),
        out_shape=jax.ShapeDtypeStruct((N_NODES, D), jnp.float32),
    )(x, norm)

    pad = E_PAD - N_EDGES
    src2d = jnp.concatenate(
        [edge_index[0], jnp.zeros((pad,), jnp.int32)]).reshape(-1, CHUNK)
    dst2d = jnp.concatenate(
        [edge_index[1], jnp.full((pad,), DUMP_ROW, jnp.int32)]).reshape(-1, CHUNK)

    mesh = plsc.VectorSubcoreMesh(core_axis_name="c", subcore_axis_name="s",
                                  num_cores=NC, num_subcores=NS)
    partials = pl.kernel(
        _edge_agg_body,
        out_type=jax.ShapeDtypeStruct((NC, ACC_ROWS, D), jnp.float32),
        mesh=mesh,
        scratch_types=[
            pltpu.VMEM((HALF, CHUNK), jnp.int32),
            pltpu.VMEM((HALF, CHUNK), jnp.int32),
            pltpu.VMEM((2, CHUNK, D), jnp.float32),
            pltpu.VMEM((16, D), jnp.float32),
            pltpu.VMEM_SHARED((ACC_ROWS, D), jnp.float32),
            pltpu.SemaphoreType.DMA,
            pltpu.SemaphoreType.DMA,
            pltpu.SemaphoreType.DMA,
            pltpu.SemaphoreType.DMA,
        ],
    )(h1, src2d, dst2d)

    p = partials[:, :N_NODES, :]

    h3, colsum, colsumsq = pl.pallas_call(
        _fc_body,
        grid=(NBLK,),
        in_specs=[pl.BlockSpec((NC, BLK, D), lambda i: (0, i, 0)),
                  pl.BlockSpec((BLK, 1), lambda i: (i, 0)),
                  pl.BlockSpec((D, D), lambda i: (0, 0)),
                  pl.BlockSpec((1, D), lambda i: (0, 0))],
        out_specs=[pl.BlockSpec((BLK, D), lambda i: (i, 0)),
                   pl.BlockSpec((1, D), lambda i: (0, 0)),
                   pl.BlockSpec((1, D), lambda i: (0, 0))],
        out_shape=[jax.ShapeDtypeStruct((N_NODES, D), jnp.float32),
                   jax.ShapeDtypeStruct((1, D), jnp.float32),
                   jax.ShapeDtypeStruct((1, D), jnp.float32)],
    )(p, norm, W, b.reshape(1, D))

    gid3 = graph_ids.reshape(NBLK, 1, BLK)
    hbn, phis = pl.pallas_call(
        _bn_body,
        grid=(NBLK,),
        in_specs=[pl.BlockSpec((BLK, D), lambda i: (i, 0)),
                  pl.BlockSpec((1, D), lambda i: (0, 0)),
                  pl.BlockSpec((1, D), lambda i: (0, 0)),
                  pl.BlockSpec((1, D), lambda i: (0, 0)),
                  pl.BlockSpec((1, D), lambda i: (0, 0)),
                  pl.BlockSpec((1, 1, BLK), lambda i: (i, 0, 0))],
        out_specs=[pl.BlockSpec((BLK, D), lambda i: (i, 0)),
                   pl.BlockSpec((NUM_GRAPHS, D), lambda i: (0, 0))],
        out_shape=[jax.ShapeDtypeStruct((N_NODES, D), jnp.float32),
                   jax.ShapeDtypeStruct((NUM_GRAPHS, D), jnp.float32)],
    )(h3, colsum, colsumsq, gamma.reshape(1, D), beta.reshape(1, D), gid3)

    return (hbn, phis)
